# baseline (speedup 1.0000x reference)
import functools

import jax
import jax.numpy as jnp
from jax import lax
from jax.experimental import pallas as pl
from jax.experimental.pallas import tpu as pltpu
from jax.experimental.pallas import tpu_sc as plsc

_L = 1024
_D = 256
_M = 4096
_NBINS = 256
_LP = _L + 8

_HI = jax.lax.Precision.DEFAULT


def _dot(a, b):
    return jnp.dot(a, b, precision=_HI, preferred_element_type=jnp.float32)


def _conv3(xb, w_ref):
    y0 = _dot(xb, w_ref[0])
    y1 = _dot(xb, w_ref[1])
    y2 = _dot(xb, w_ref[2])
    z = jnp.zeros((1, xb.shape[1]), jnp.float32)
    y0s = jnp.concatenate([z, y0[:-1, :]], axis=0)
    y2s = jnp.concatenate([y2[1:, :], z], axis=0)
    return y0s + y1 + y2s


def _rsum(a):
    ones = jnp.ones((a.shape[1], 1), jnp.float32)
    return _dot(a, ones)


def _count_le_bf16(sel_bool):
    sel = sel_bool.astype(jnp.bfloat16)
    ones = jnp.ones((sel.shape[1], 1), jnp.bfloat16)
    return jnp.dot(sel, ones, preferred_element_type=jnp.float32)


def _ln(h, g, b):
    m = jnp.mean(h, axis=-1, keepdims=True)
    e = h - m
    v = jnp.mean(e * e, axis=-1, keepdims=True)
    return e / jnp.sqrt(v + 1e-5) * g + b


def _predict(xb, mask_keep, w1, b1, g1, be1, w2, b2, g2, be2, wl, bl):
    h = jax.nn.relu(_conv3(xb, w1) + b1[...])
    h = _ln(h, g1[...], be1[...])
    h = jax.nn.relu(_conv3(h, w2) + b2[...])
    h = _ln(h, g2[...], be2[...])
    o = _dot(h, wl[...]) + bl[...]
    return o * mask_keep


def _emb_add(vcol, bins_row, emb_ref):
    cnt = _count_le_bf16(bins_row < vcol)
    lane = jax.lax.broadcasted_iota(jnp.int32, (vcol.shape[0], _NBINS), 1)
    onehot = (cnt == lane.astype(jnp.float32)).astype(jnp.float32)
    return _dot(onehot, emb_ref[...])


def _body(x_ref, maskf_ref, durf_ref, pt_ref, et_ref, binsp_ref, binse_ref,
          pemb_ref, eemb_ref, *rest):
    prm, (x3_ref, gidx_ref, ld_ref, pd_ref, ed_ref, tot_ref) = rest[:30], rest[30:]
    dp_p, pp_p, ep_p = prm[0:10], prm[10:20], prm[20:30]

    xb = x_ref[0]
    mask_keep = 1.0 - maskf_ref[0]

    ld_ref[0] = _predict(xb, mask_keep, *dp_p)
    pd_ref[0] = _predict(xb, mask_keep, *pp_p)
    x2 = xb + _emb_add(pt_ref[0], binsp_ref[...], pemb_ref)
    ed_ref[0] = _predict(x2, mask_keep, *ep_p)
    x3 = x2 + _emb_add(et_ref[0], binse_ref[...], eemb_ref)

    x3_ref[0, : _L, :] = x3
    x3_ref[0, _L:, :] = jnp.zeros((_LP - _L, _D), jnp.float32)

    d_row = durf_ref[0]
    ii = jax.lax.broadcasted_iota(jnp.int32, (_L, _L), 0)
    jj = jax.lax.broadcasted_iota(jnp.int32, (_L, _L), 1)
    cum = jnp.dot(d_row.astype(jnp.bfloat16),
                  (ii <= jj).astype(jnp.bfloat16),
                  preferred_element_type=jnp.float32)
    tot_ref[...] = jnp.sum(d_row).astype(jnp.int32).reshape(1, 1, 1)

    base = pl.program_id(0) * _LP
    chunk = 1024
    for c in range(_M // chunk):
        tcol = (jax.lax.broadcasted_iota(jnp.int32, (chunk, 1), 0)
                + c * chunk).astype(jnp.float32)
        cnt = _count_le_bf16(cum <= tcol)
        gidx_ref[0, pl.ds(c * chunk, chunk), :] = cnt.astype(jnp.int32) + base


@jax.jit
def _run_tc(x, maskf, durf, pt, et, binsp, binse, pemb, eemb, prm):
    b = x.shape[0]
    row_spec = pl.BlockSpec((1, _L, 1), lambda i: (i, 0, 0))
    full2 = lambda a: pl.BlockSpec(a.shape, lambda i: (0,) * a.ndim)
    in_specs = [
        pl.BlockSpec((1, _L, _D), lambda i: (i, 0, 0)),
        row_spec,
        pl.BlockSpec((1, 1, _L), lambda i: (i, 0, 0)),
        row_spec,
        row_spec,
        full2(binsp), full2(binse), full2(pemb), full2(eemb),
    ] + [full2(p) for p in prm]
    out_specs = (
        pl.BlockSpec((1, _LP, _D), lambda i: (i, 0, 0)),
        pl.BlockSpec((1, _M, 1), lambda i: (i, 0, 0)),
        row_spec, row_spec, row_spec,
        pl.BlockSpec((1, 1, 1), lambda i: (i, 0, 0)),
    )
    out_shape = (
        jax.ShapeDtypeStruct((b, _LP, _D), jnp.float32),
        jax.ShapeDtypeStruct((b, _M, 1), jnp.int32),
        jax.ShapeDtypeStruct((b, _L, 1), jnp.float32),
        jax.ShapeDtypeStruct((b, _L, 1), jnp.float32),
        jax.ShapeDtypeStruct((b, _L, 1), jnp.float32),
        jax.ShapeDtypeStruct((b, 1, 1), jnp.int32),
    )
    return pl.pallas_call(
        _body,
        grid=(b,),
        in_specs=in_specs,
        out_specs=out_specs,
        out_shape=out_shape,
    )(x, maskf, durf, pt, et, binsp, binse, pemb, eemb, *prm)


_CHUNK = 128


@jax.jit
def _sc_gather(table, gidx2):
    info = plsc.get_sparse_core_info()
    nc, ns = info.num_cores, info.num_subcores
    nw = nc * ns
    total_chunks = gidx2.shape[0]
    n_chunks = total_chunks // nw
    rows_total = total_chunks * _CHUNK
    mesh = plsc.VectorSubcoreMesh(core_axis_name="c", subcore_axis_name="s")

    @functools.partial(
        pl.kernel, mesh=mesh,
        out_type=jax.ShapeDtypeStruct((rows_total, _D), jnp.float32),
        scratch_types=[
            pltpu.VMEM((n_chunks, _CHUNK), jnp.int32),
            pltpu.VMEM((3, _CHUNK, _D), jnp.float32),
            pltpu.SemaphoreType.DMA,
            pltpu.SemaphoreType.DMA,
            pltpu.SemaphoreType.DMA,
            pltpu.SemaphoreType.DMA,
            pltpu.SemaphoreType.DMA,
            pltpu.SemaphoreType.DMA,
        ],
    )
    def k(table_hbm, gidx_hbm, out_hbm, idx_v, rows_v, g0, g1, g2, s0, s1, s2):
        wid = lax.axis_index("s") * nc + lax.axis_index("c")
        gs, ss = [g0, g1, g2], [s0, s1, s2]
        gat = [None, None, None]
        sca = [None, None, None]

        base_chunk = wid * n_chunks
        pltpu.sync_copy(gidx_hbm.at[pl.ds(base_chunk, n_chunks)], idx_v)

        def fire(g):
            bb = g % 3
            if sca[bb] is not None:
                sca[bb].wait()
                sca[bb] = None
            gat[bb] = pltpu.async_copy(
                table_hbm.at[idx_v.at[g]], rows_v.at[bb], gs[bb])

        fire(0)
        if n_chunks > 1:
            fire(1)
        for g in range(n_chunks):
            bb = g % 3
            gat[bb].wait()
            gat[bb] = None
            if g + 2 < n_chunks:
                fire(g + 2)
            sca[bb] = pltpu.async_copy(
                rows_v.at[bb],
                out_hbm.at[pl.ds((base_chunk + g) * _CHUNK, _CHUNK)],
                ss[bb])
        for s in sca:
            if s is not None:
                s.wait()

    return k(table, gidx2)


_PKEYS = ("w1", "b1", "g1", "be1", "w2", "b2", "g2", "be2", "wl", "bl")


def _flat_params(p):
    out = []
    for k in _PKEYS:
        a = p[k]
        if k in ("b1", "g1", "be1", "b2", "g2", "be2"):
            a = a.reshape(1, _D)
        elif k == "bl":
            a = a.reshape(1, 1)
        out.append(a)
    return out


def kernel(x, src_mask, duration_target, pitch_target, energy_target, max_len,
           dp, pp, ep, pitch_bins, energy_bins, pitch_emb, energy_emb):
    b, l, d = x.shape
    maskf = src_mask.astype(jnp.float32).reshape(b, l, 1)
    durf = duration_target.astype(jnp.float32).reshape(b, 1, l)
    pt = pitch_target.reshape(b, l, 1)
    et = energy_target.reshape(b, l, 1)
    pad = jnp.full((1,), jnp.inf, jnp.float32)
    binsp = jnp.concatenate([pitch_bins, pad]).reshape(1, _NBINS)
    binse = jnp.concatenate([energy_bins, pad]).reshape(1, _NBINS)
    prm = _flat_params(dp) + _flat_params(pp) + _flat_params(ep)

    nh = 2
    hb = b // nh
    outs, lds, pds, eds, tots = [], [], [], [], []
    for h in range(nh):
        s = slice(h * hb, (h + 1) * hb)
        x3p, gidx, ld, pd, ed, tot = _run_tc(
            x[s], maskf[s], durf[s], pt[s], et[s], binsp, binse,
            pitch_emb, energy_emb, prm)
        gidx2 = gidx.reshape((hb * _M) // _CHUNK, _CHUNK)
        outs.append(_sc_gather(x3p.reshape(hb * _LP, d), gidx2)
                    .reshape(hb, _M, d))
        lds.append(ld)
        pds.append(pd)
        eds.append(ed)
        tots.append(tot)

    out = jnp.concatenate(outs, axis=0)
    ld = jnp.concatenate(lds, axis=0)
    pd = jnp.concatenate(pds, axis=0)
    ed = jnp.concatenate(eds, axis=0)
    tot = jnp.concatenate(tots, axis=0)
    mel_len = jnp.minimum(tot.reshape(b), max_len)
    return (out, ld.reshape(b, l), pd.reshape(b, l), ed.reshape(b, l),
            duration_target, mel_len)


# speedup vs baseline: 1.2261x; 1.2261x over previous
import functools

import jax
import jax.numpy as jnp
from jax import lax
from jax.experimental import pallas as pl
from jax.experimental.pallas import tpu as pltpu
from jax.experimental.pallas import tpu_sc as plsc

_L = 1024
_D = 256
_M = 4096
_NBINS = 256
_LP = _L + 8

_HI = jax.lax.Precision.DEFAULT


def _dot(a, b):
    return jnp.dot(a, b, precision=_HI, preferred_element_type=jnp.float32)


def _conv3(xb, w_ref):
    y0 = _dot(xb, w_ref[0])
    y1 = _dot(xb, w_ref[1])
    y2 = _dot(xb, w_ref[2])
    z = jnp.zeros((1, xb.shape[1]), jnp.float32)
    y0s = jnp.concatenate([z, y0[:-1, :]], axis=0)
    y2s = jnp.concatenate([y2[1:, :], z], axis=0)
    return y0s + y1 + y2s


def _rsum(a):
    ones = jnp.ones((a.shape[1], 1), jnp.float32)
    return _dot(a, ones)


def _count_le_bf16(sel_bool):
    sel = sel_bool.astype(jnp.bfloat16)
    ones = jnp.ones((sel.shape[1], 1), jnp.bfloat16)
    return jnp.dot(sel, ones, preferred_element_type=jnp.float32)


def _ln(h, g, b):
    m = jnp.mean(h, axis=-1, keepdims=True)
    e = h - m
    v = jnp.mean(e * e, axis=-1, keepdims=True)
    return e * jax.lax.rsqrt(v + 1e-5) * g + b


def _predict(xb, mask_keep, w1, b1, g1, be1, w2, b2, g2, be2, wl, bl):
    h = jax.nn.relu(_conv3(xb, w1) + b1[...])
    h = _ln(h, g1[...], be1[...])
    h = jax.nn.relu(_conv3(h, w2) + b2[...])
    h = _ln(h, g2[...], be2[...])
    o = _dot(h, wl[...]) + bl[...]
    return o * mask_keep


def _emb_add(vcol, bins_row, emb_ref):
    cnt = _count_le_bf16(bins_row < vcol)
    lane = jax.lax.broadcasted_iota(jnp.int32, (vcol.shape[0], _NBINS), 1)
    onehot = (cnt == lane.astype(jnp.float32)).astype(jnp.float32)
    return _dot(onehot, emb_ref[...])


def _body(x_ref, maskf_ref, durf_ref, pt_ref, et_ref, binsp_ref, binse_ref,
          pemb_ref, eemb_ref, *rest):
    prm, (x3_ref, gidx_ref, ld_ref, pd_ref, ed_ref, tot_ref) = rest[:30], rest[30:]
    dp_p, pp_p, ep_p = prm[0:10], prm[10:20], prm[20:30]

    xb = x_ref[0]
    mask_keep = 1.0 - maskf_ref[0]

    ld_ref[0] = _predict(xb, mask_keep, *dp_p)
    pd_ref[0] = _predict(xb, mask_keep, *pp_p)
    x2 = xb + _emb_add(pt_ref[0], binsp_ref[...], pemb_ref)
    ed_ref[0] = _predict(x2, mask_keep, *ep_p)
    x3 = x2 + _emb_add(et_ref[0], binse_ref[...], eemb_ref)

    x3_ref[0, : _L, :] = x3
    x3_ref[0, _L:, :] = jnp.zeros((_LP - _L, _D), jnp.float32)

    d_row = durf_ref[0]
    ii = jax.lax.broadcasted_iota(jnp.int32, (_L, _L), 0)
    jj = jax.lax.broadcasted_iota(jnp.int32, (_L, _L), 1)
    cum = jnp.dot(d_row.astype(jnp.bfloat16),
                  (ii <= jj).astype(jnp.bfloat16),
                  preferred_element_type=jnp.float32)
    tot_ref[...] = jnp.sum(d_row).astype(jnp.int32).reshape(1, 1, 1)

    base = pl.program_id(0) * _LP
    chunk = 1024
    for c in range(_M // chunk):
        tcol = (jax.lax.broadcasted_iota(jnp.int32, (chunk, 1), 0)
                + c * chunk).astype(jnp.float32)
        cnt = _count_le_bf16(cum <= tcol)
        gidx_ref[0, pl.ds(c * chunk, chunk), :] = cnt.astype(jnp.int32) + base


@jax.jit
def _run_tc(x, maskf, durf, pt, et, binsp, binse, pemb, eemb, prm):
    b = x.shape[0]
    row_spec = pl.BlockSpec((1, _L, 1), lambda i: (i, 0, 0))
    full2 = lambda a: pl.BlockSpec(a.shape, lambda i: (0,) * a.ndim)
    in_specs = [
        pl.BlockSpec((1, _L, _D), lambda i: (i, 0, 0)),
        row_spec,
        pl.BlockSpec((1, 1, _L), lambda i: (i, 0, 0)),
        row_spec,
        row_spec,
        full2(binsp), full2(binse), full2(pemb), full2(eemb),
    ] + [full2(p) for p in prm]
    out_specs = (
        pl.BlockSpec((1, _LP, _D), lambda i: (i, 0, 0)),
        pl.BlockSpec((1, _M, 1), lambda i: (i, 0, 0)),
        row_spec, row_spec, row_spec,
        pl.BlockSpec((1, 1, 1), lambda i: (i, 0, 0)),
    )
    out_shape = (
        jax.ShapeDtypeStruct((b, _LP, _D), jnp.float32),
        jax.ShapeDtypeStruct((b, _M, 1), jnp.int32),
        jax.ShapeDtypeStruct((b, _L, 1), jnp.float32),
        jax.ShapeDtypeStruct((b, _L, 1), jnp.float32),
        jax.ShapeDtypeStruct((b, _L, 1), jnp.float32),
        jax.ShapeDtypeStruct((b, 1, 1), jnp.int32),
    )
    return pl.pallas_call(
        _body,
        grid=(b,),
        in_specs=in_specs,
        out_specs=out_specs,
        out_shape=out_shape,
    )(x, maskf, durf, pt, et, binsp, binse, pemb, eemb, *prm)


_CHUNK = 128


@jax.jit
def _sc_gather(table, gidx2):
    info = plsc.get_sparse_core_info()
    nc, ns = info.num_cores, info.num_subcores
    nw = nc * ns
    total_chunks = gidx2.shape[0]
    n_chunks = total_chunks // nw
    rows_total = total_chunks * _CHUNK
    mesh = plsc.VectorSubcoreMesh(core_axis_name="c", subcore_axis_name="s")

    @functools.partial(
        pl.kernel, mesh=mesh,
        out_type=jax.ShapeDtypeStruct((rows_total, _D), jnp.float32),
        scratch_types=[
            pltpu.VMEM((n_chunks, _CHUNK), jnp.int32),
            pltpu.VMEM((3, _CHUNK, _D), jnp.float32),
            pltpu.SemaphoreType.DMA,
            pltpu.SemaphoreType.DMA,
            pltpu.SemaphoreType.DMA,
            pltpu.SemaphoreType.DMA,
            pltpu.SemaphoreType.DMA,
            pltpu.SemaphoreType.DMA,
        ],
    )
    def k(table_hbm, gidx_hbm, out_hbm, idx_v, rows_v, g0, g1, g2, s0, s1, s2):
        cc = lax.axis_index("c")
        ss_i = lax.axis_index("s")
        wid = ss_i * nc + (cc + ss_i) % nc
        gs, ss = [g0, g1, g2], [s0, s1, s2]
        gat = [None, None, None]
        sca = [None, None, None]

        base_chunk = wid * n_chunks
        pltpu.sync_copy(gidx_hbm.at[pl.ds(base_chunk, n_chunks)], idx_v)

        def fire(g):
            bb = g % 3
            if sca[bb] is not None:
                sca[bb].wait()
                sca[bb] = None
            gat[bb] = pltpu.async_copy(
                table_hbm.at[idx_v.at[g]], rows_v.at[bb], gs[bb])

        fire(0)
        if n_chunks > 1:
            fire(1)
        for g in range(n_chunks):
            bb = g % 3
            gat[bb].wait()
            gat[bb] = None
            if g + 2 < n_chunks:
                fire(g + 2)
            sca[bb] = pltpu.async_copy(
                rows_v.at[bb],
                out_hbm.at[pl.ds((base_chunk + g) * _CHUNK, _CHUNK)],
                ss[bb])
        for s in sca:
            if s is not None:
                s.wait()

    return k(table, gidx2)


_PKEYS = ("w1", "b1", "g1", "be1", "w2", "b2", "g2", "be2", "wl", "bl")


def _flat_params(p):
    out = []
    for k in _PKEYS:
        a = p[k]
        if k in ("b1", "g1", "be1", "b2", "g2", "be2"):
            a = a.reshape(1, _D)
        elif k == "bl":
            a = a.reshape(1, 1)
        out.append(a)
    return out


def kernel(x, src_mask, duration_target, pitch_target, energy_target, max_len,
           dp, pp, ep, pitch_bins, energy_bins, pitch_emb, energy_emb):
    b, l, d = x.shape
    maskf = src_mask.astype(jnp.float32).reshape(b, l, 1)
    durf = duration_target.astype(jnp.float32).reshape(b, 1, l)
    pt = pitch_target.reshape(b, l, 1)
    et = energy_target.reshape(b, l, 1)
    pad = jnp.full((1,), jnp.inf, jnp.float32)
    binsp = jnp.concatenate([pitch_bins, pad]).reshape(1, _NBINS)
    binse = jnp.concatenate([energy_bins, pad]).reshape(1, _NBINS)
    prm = _flat_params(dp) + _flat_params(pp) + _flat_params(ep)

    x3p, gidx, ld, pd, ed, tot = _run_tc(x, maskf, durf, pt, et, binsp, binse,
                                         pitch_emb, energy_emb, prm)

    gidx2 = gidx.reshape((b * _M) // _CHUNK, _CHUNK)
    out = _sc_gather(x3p.reshape(b * _LP, d), gidx2).reshape(b, _M, d)

    mel_len = jnp.minimum(tot.reshape(b), max_len)
    return (out, ld.reshape(b, l), pd.reshape(b, l), ed.reshape(b, l),
            duration_target, mel_len)


# speedup vs baseline: 1.2331x; 1.0057x over previous
import functools

import jax
import jax.numpy as jnp
from jax import lax
from jax.experimental import pallas as pl
from jax.experimental.pallas import tpu as pltpu
from jax.experimental.pallas import tpu_sc as plsc

_L = 1024
_D = 256
_M = 4096
_NBINS = 256
_LP = _L + 8

_HI = jax.lax.Precision.DEFAULT


def _dot(a, b):
    return jnp.dot(a, b, precision=_HI, preferred_element_type=jnp.float32)


def _conv3(xb, w_ref):
    y0 = _dot(xb, w_ref[0])
    y1 = _dot(xb, w_ref[1])
    y2 = _dot(xb, w_ref[2])
    z = jnp.zeros((1, xb.shape[1]), jnp.float32)
    y0s = jnp.concatenate([z, y0[:-1, :]], axis=0)
    y2s = jnp.concatenate([y2[1:, :], z], axis=0)
    return y0s + y1 + y2s


def _rsum(a):
    ones = jnp.ones((a.shape[1], 1), jnp.float32)
    return _dot(a, ones)


def _count_le_bf16(sel_bool):
    sel = sel_bool.astype(jnp.bfloat16)
    ones = jnp.ones((sel.shape[1], 1), jnp.bfloat16)
    return jnp.dot(sel, ones, preferred_element_type=jnp.float32)


def _ln(h, g, b):
    m = jnp.mean(h, axis=-1, keepdims=True)
    e = h - m
    v = jnp.mean(e * e, axis=-1, keepdims=True)
    return e * jax.lax.rsqrt(v + 1e-5) * g + b


def _predict(xb, mask_keep, w1, b1, g1, be1, w2, b2, g2, be2, wl, bl):
    h = jax.nn.relu(_conv3(xb, w1) + b1[...])
    h = _ln(h, g1[...], be1[...])
    h = jax.nn.relu(_conv3(h, w2) + b2[...])
    h = _ln(h, g2[...], be2[...])
    o = _dot(h, wl[...]) + bl[...]
    return o * mask_keep


def _emb_add(vcol, bins_row, emb_ref):
    cnt = _count_le_bf16(bins_row < vcol)
    lane = jax.lax.broadcasted_iota(jnp.int32, (vcol.shape[0], _NBINS), 1)
    onehot = (cnt == lane.astype(jnp.float32)).astype(jnp.float32)
    return _dot(onehot, emb_ref[...])


def _body(x_ref, maskf_ref, durf_ref, pt_ref, et_ref, binsp_ref, binse_ref,
          pemb_ref, eemb_ref, *rest):
    prm, (x3_ref, gidx_ref, ld_ref, pd_ref, ed_ref, tot_ref) = rest[:30], rest[30:]
    dp_p, pp_p, ep_p = prm[0:10], prm[10:20], prm[20:30]

    xb = x_ref[0]
    mask_keep = 1.0 - maskf_ref[0]

    ld_ref[0] = _predict(xb, mask_keep, *dp_p)
    pd_ref[0] = _predict(xb, mask_keep, *pp_p)
    x2 = xb + _emb_add(pt_ref[0], binsp_ref[...], pemb_ref)
    ed_ref[0] = _predict(x2, mask_keep, *ep_p)
    x3 = x2 + _emb_add(et_ref[0], binse_ref[...], eemb_ref)

    x3_ref[0, : _L, :] = x3
    x3_ref[0, _L:, :] = jnp.zeros((_LP - _L, _D), jnp.float32)

    d_row = durf_ref[0]
    ii = jax.lax.broadcasted_iota(jnp.int32, (_L, _L), 0)
    jj = jax.lax.broadcasted_iota(jnp.int32, (_L, _L), 1)
    cum = jnp.dot(d_row.astype(jnp.bfloat16),
                  (ii <= jj).astype(jnp.bfloat16),
                  preferred_element_type=jnp.float32)
    tot_ref[...] = jnp.sum(d_row).astype(jnp.int32).reshape(1, 1, 1)

    base = pl.program_id(0) * _LP
    chunk = 1024
    for c in range(_M // chunk):
        tcol = (jax.lax.broadcasted_iota(jnp.int32, (chunk, 1), 0)
                + c * chunk).astype(jnp.float32)
        cnt = _count_le_bf16(cum <= tcol)
        gidx_ref[0, pl.ds(c * chunk, chunk), :] = cnt.astype(jnp.int32) + base


@jax.jit
def _run_tc(x, maskf, durf, pt, et, binsp, binse, pemb, eemb, prm):
    b = x.shape[0]
    row_spec = pl.BlockSpec((1, _L, 1), lambda i: (i, 0, 0))
    full2 = lambda a: pl.BlockSpec(a.shape, lambda i: (0,) * a.ndim)
    in_specs = [
        pl.BlockSpec((1, _L, _D), lambda i: (i, 0, 0)),
        row_spec,
        pl.BlockSpec((1, 1, _L), lambda i: (i, 0, 0)),
        row_spec,
        row_spec,
        full2(binsp), full2(binse), full2(pemb), full2(eemb),
    ] + [full2(p) for p in prm]
    out_specs = (
        pl.BlockSpec((1, _LP, _D), lambda i: (i, 0, 0)),
        pl.BlockSpec((1, _M, 1), lambda i: (i, 0, 0)),
        row_spec, row_spec, row_spec,
        pl.BlockSpec((1, 1, 1), lambda i: (i, 0, 0)),
    )
    out_shape = (
        jax.ShapeDtypeStruct((b, _LP, _D), jnp.float32),
        jax.ShapeDtypeStruct((b, _M, 1), jnp.int32),
        jax.ShapeDtypeStruct((b, _L, 1), jnp.float32),
        jax.ShapeDtypeStruct((b, _L, 1), jnp.float32),
        jax.ShapeDtypeStruct((b, _L, 1), jnp.float32),
        jax.ShapeDtypeStruct((b, 1, 1), jnp.int32),
    )
    return pl.pallas_call(
        _body,
        grid=(b,),
        in_specs=in_specs,
        out_specs=out_specs,
        out_shape=out_shape,
    )(x, maskf, durf, pt, et, binsp, binse, pemb, eemb, *prm)


_CHUNK = 64
_NBUF = 6


@jax.jit
def _sc_gather(table, gidx2):
    info = plsc.get_sparse_core_info()
    nc, ns = info.num_cores, info.num_subcores
    nw = nc * ns
    total_chunks = gidx2.shape[0]
    n_chunks = total_chunks // nw
    rows_total = total_chunks * _CHUNK
    mesh = plsc.VectorSubcoreMesh(core_axis_name="c", subcore_axis_name="s")

    @functools.partial(
        pl.kernel, mesh=mesh,
        out_type=jax.ShapeDtypeStruct((rows_total, _D), jnp.float32),
        scratch_types=[
            pltpu.VMEM((n_chunks, _CHUNK), jnp.int32),
            pltpu.VMEM((_NBUF, _CHUNK, _D), jnp.float32),
        ] + [pltpu.SemaphoreType.DMA] * (2 * _NBUF),
    )
    def k(table_hbm, gidx_hbm, out_hbm, idx_v, rows_v, *sems):
        cc = lax.axis_index("c")
        ss_i = lax.axis_index("s")
        wid = ss_i * nc + (cc + ss_i) % nc
        gs, ss = list(sems[:_NBUF]), list(sems[_NBUF:])
        gat = [None] * _NBUF
        sca = [None] * _NBUF

        base_chunk = wid * n_chunks
        pltpu.sync_copy(gidx_hbm.at[pl.ds(base_chunk, n_chunks)], idx_v)

        def fire(g):
            bb = g % _NBUF
            if sca[bb] is not None:
                sca[bb].wait()
                sca[bb] = None
            gat[bb] = pltpu.async_copy(
                table_hbm.at[idx_v.at[g]], rows_v.at[bb], gs[bb])

        depth = min(_NBUF - 1, n_chunks)
        for g in range(depth):
            fire(g)
        for g in range(n_chunks):
            bb = g % _NBUF
            gat[bb].wait()
            gat[bb] = None
            if g + depth < n_chunks:
                fire(g + depth)
            sca[bb] = pltpu.async_copy(
                rows_v.at[bb],
                out_hbm.at[pl.ds((base_chunk + g) * _CHUNK, _CHUNK)],
                ss[bb])
        for s in sca:
            if s is not None:
                s.wait()

    return k(table, gidx2)


_PKEYS = ("w1", "b1", "g1", "be1", "w2", "b2", "g2", "be2", "wl", "bl")


def _flat_params(p):
    out = []
    for k in _PKEYS:
        a = p[k]
        if k in ("b1", "g1", "be1", "b2", "g2", "be2"):
            a = a.reshape(1, _D)
        elif k == "bl":
            a = a.reshape(1, 1)
        out.append(a)
    return out


def kernel(x, src_mask, duration_target, pitch_target, energy_target, max_len,
           dp, pp, ep, pitch_bins, energy_bins, pitch_emb, energy_emb):
    b, l, d = x.shape
    maskf = src_mask.astype(jnp.float32).reshape(b, l, 1)
    durf = duration_target.astype(jnp.float32).reshape(b, 1, l)
    pt = pitch_target.reshape(b, l, 1)
    et = energy_target.reshape(b, l, 1)
    pad = jnp.full((1,), jnp.inf, jnp.float32)
    binsp = jnp.concatenate([pitch_bins, pad]).reshape(1, _NBINS)
    binse = jnp.concatenate([energy_bins, pad]).reshape(1, _NBINS)
    prm = _flat_params(dp) + _flat_params(pp) + _flat_params(ep)

    x3p, gidx, ld, pd, ed, tot = _run_tc(x, maskf, durf, pt, et, binsp, binse,
                                         pitch_emb, energy_emb, prm)

    gidx2 = gidx.reshape((b * _M) // _CHUNK, _CHUNK)
    out = _sc_gather(x3p.reshape(b * _LP, d), gidx2).reshape(b, _M, d)

    mel_len = jnp.minimum(tot.reshape(b), max_len)
    return (out, ld.reshape(b, l), pd.reshape(b, l), ed.reshape(b, l),
            duration_target, mel_len)


# speedup vs baseline: 1.2331x; 1.0000x over previous
import functools

import jax
import jax.numpy as jnp
from jax import lax
from jax.experimental import pallas as pl
from jax.experimental.pallas import tpu as pltpu
from jax.experimental.pallas import tpu_sc as plsc

_L = 1024
_D = 256
_M = 4096
_NBINS = 256
_LP = _L + 8

_HI = jax.lax.Precision.DEFAULT


def _dot(a, b):
    return jnp.dot(a, b, precision=_HI, preferred_element_type=jnp.float32)


def _conv3(xb, w_ref):
    y0 = _dot(xb, w_ref[0])
    y1 = _dot(xb, w_ref[1])
    y2 = _dot(xb, w_ref[2])
    z = jnp.zeros((1, xb.shape[1]), jnp.float32)
    y0s = jnp.concatenate([z, y0[:-1, :]], axis=0)
    y2s = jnp.concatenate([y2[1:, :], z], axis=0)
    return y0s + y1 + y2s


def _count_le_bf16(sel_bool):
    sel = sel_bool.astype(jnp.bfloat16)
    ones = jnp.ones((sel.shape[1], 1), jnp.bfloat16)
    return jnp.dot(sel, ones, preferred_element_type=jnp.float32)


def _ln(h, g, b):
    m = jnp.mean(h, axis=-1, keepdims=True)
    e = h - m
    v = jnp.mean(e * e, axis=-1, keepdims=True)
    return e * jax.lax.rsqrt(v + 1e-5) * g + b


def _predict(xb, mask_keep, w1, b1, g1, be1, w2, b2, g2, be2, wl, bl):
    h = jax.nn.relu(_conv3(xb, w1) + b1[...])
    h = _ln(h, g1[...], be1[...])
    h = jax.nn.relu(_conv3(h, w2) + b2[...])
    h = _ln(h, g2[...], be2[...])
    o = _dot(h, wl[...]) + bl[...]
    return o * mask_keep


def _emb_add(vcol, bins_row, emb_ref):
    cnt = _count_le_bf16(bins_row < vcol)
    lane = jax.lax.broadcasted_iota(jnp.int32, (vcol.shape[0], _NBINS), 1)
    onehot = (cnt == lane.astype(jnp.float32)).astype(jnp.float32)
    return _dot(onehot, emb_ref[...])


def _body(x_ref, maskf_ref, durf_ref, pt_ref, et_ref, binsp_ref, binse_ref,
          pemb_ref, eemb_ref, *rest):
    prm, (x3_ref, gidx_ref, ld_ref, pd_ref, ed_ref, tot_ref) = rest[:30], rest[30:]
    dp_p, pp_p, ep_p = prm[0:10], prm[10:20], prm[20:30]

    xb = x_ref[0]
    mask_keep = 1.0 - maskf_ref[0]

    ld_ref[0] = _predict(xb, mask_keep, *dp_p)
    pd_ref[0] = _predict(xb, mask_keep, *pp_p)
    x2 = xb + _emb_add(pt_ref[0], binsp_ref[...], pemb_ref)
    ed_ref[0] = _predict(x2, mask_keep, *ep_p)
    x3 = x2 + _emb_add(et_ref[0], binse_ref[...], eemb_ref)

    x3_ref[0, : _L, :] = x3
    x3_ref[0, _L:, :] = jnp.zeros((_LP - _L, _D), jnp.float32)

    d_row = durf_ref[0]
    ii = jax.lax.broadcasted_iota(jnp.int32, (_L, _L), 0)
    jj = jax.lax.broadcasted_iota(jnp.int32, (_L, _L), 1)
    cum = jnp.dot(d_row.astype(jnp.bfloat16),
                  (ii <= jj).astype(jnp.bfloat16),
                  preferred_element_type=jnp.float32)
    tot_ref[...] = jnp.sum(d_row).astype(jnp.int32).reshape(1, 1, 1)

    base = pl.program_id(0) * _LP
    chunk = 1024
    for c in range(_M // chunk):
        tcol = (jax.lax.broadcasted_iota(jnp.int32, (chunk, 1), 0)
                + c * chunk).astype(jnp.float32)
        cnt = _count_le_bf16(cum <= tcol)
        gidx_ref[0, pl.ds(c * chunk, chunk), :] = cnt.astype(jnp.int32) + base


@jax.jit
def _run_tc(x, maskf, durf, pt, et, binsp, binse, pemb, eemb, prm):
    b = x.shape[0]
    row_spec = pl.BlockSpec((1, _L, 1), lambda i: (i, 0, 0))
    full2 = lambda a: pl.BlockSpec(a.shape, lambda i: (0,) * a.ndim)
    in_specs = [
        pl.BlockSpec((1, _L, _D), lambda i: (i, 0, 0)),
        row_spec,
        pl.BlockSpec((1, 1, _L), lambda i: (i, 0, 0)),
        row_spec,
        row_spec,
        full2(binsp), full2(binse), full2(pemb), full2(eemb),
    ] + [full2(p) for p in prm]
    out_specs = (
        pl.BlockSpec((1, _LP, _D), lambda i: (i, 0, 0)),
        pl.BlockSpec((1, _M, 1), lambda i: (i, 0, 0)),
        row_spec, row_spec, row_spec,
        pl.BlockSpec((1, 1, 1), lambda i: (i, 0, 0)),
    )
    out_shape = (
        jax.ShapeDtypeStruct((b, _LP, _D), jnp.float32),
        jax.ShapeDtypeStruct((b, _M, 1), jnp.int32),
        jax.ShapeDtypeStruct((b, _L, 1), jnp.float32),
        jax.ShapeDtypeStruct((b, _L, 1), jnp.float32),
        jax.ShapeDtypeStruct((b, _L, 1), jnp.float32),
        jax.ShapeDtypeStruct((b, 1, 1), jnp.int32),
    )
    return pl.pallas_call(
        _body,
        grid=(b,),
        in_specs=in_specs,
        out_specs=out_specs,
        out_shape=out_shape,
    )(x, maskf, durf, pt, et, binsp, binse, pemb, eemb, *prm)


_CHUNK = 64
_NBUF = 6


@jax.jit
def _sc_gather(table, gidx2):
    info = plsc.get_sparse_core_info()
    nc, ns = info.num_cores, info.num_subcores
    nw = nc * ns
    total_chunks = gidx2.shape[0]
    n_chunks = total_chunks // nw
    rows_total = total_chunks * _CHUNK
    mesh = plsc.VectorSubcoreMesh(core_axis_name="c", subcore_axis_name="s")

    @functools.partial(
        pl.kernel, mesh=mesh,
        out_type=jax.ShapeDtypeStruct((rows_total, _D), jnp.float32),
        scratch_types=[
            pltpu.VMEM((n_chunks, _CHUNK), jnp.int32),
            pltpu.VMEM((_NBUF, _CHUNK, _D), jnp.float32),
        ] + [pltpu.SemaphoreType.DMA] * (2 * _NBUF),
    )
    def k(table_hbm, gidx_hbm, out_hbm, idx_v, rows_v, *sems):
        cc = lax.axis_index("c")
        ss_i = lax.axis_index("s")
        wid = ss_i * nc + (cc + ss_i) % nc
        gs, ss = list(sems[:_NBUF]), list(sems[_NBUF:])
        gat = [None] * _NBUF
        sca = [None] * _NBUF

        base_chunk = wid * n_chunks
        pltpu.sync_copy(gidx_hbm.at[pl.ds(base_chunk, n_chunks)], idx_v)

        def fire(g):
            bb = g % _NBUF
            if sca[bb] is not None:
                sca[bb].wait()
                sca[bb] = None
            gat[bb] = pltpu.async_copy(
                table_hbm.at[idx_v.at[g]], rows_v.at[bb], gs[bb])

        depth = min(_NBUF - 1, n_chunks)
        for g in range(depth):
            fire(g)
        for g in range(n_chunks):
            bb = g % _NBUF
            gat[bb].wait()
            gat[bb] = None
            if g + depth < n_chunks:
                fire(g + depth)
            sca[bb] = pltpu.async_copy(
                rows_v.at[bb],
                out_hbm.at[pl.ds((base_chunk + g) * _CHUNK, _CHUNK)],
                ss[bb])
        for s in sca:
            if s is not None:
                s.wait()

    return k(table, gidx2)


_PKEYS = ("w1", "b1", "g1", "be1", "w2", "b2", "g2", "be2", "wl", "bl")


def _flat_params(p):
    out = []
    for k in _PKEYS:
        a = p[k]
        if k in ("b1", "g1", "be1", "b2", "g2", "be2"):
            a = a.reshape(1, _D)
        elif k == "bl":
            a = a.reshape(1, 1)
        out.append(a)
    return out


def kernel(x, src_mask, duration_target, pitch_target, energy_target, max_len,
           dp, pp, ep, pitch_bins, energy_bins, pitch_emb, energy_emb):
    b, l, d = x.shape
    maskf = src_mask.astype(jnp.float32).reshape(b, l, 1)
    durf = duration_target.astype(jnp.float32).reshape(b, 1, l)
    pt = pitch_target.reshape(b, l, 1)
    et = energy_target.reshape(b, l, 1)
    pad = jnp.full((1,), jnp.inf, jnp.float32)
    binsp = jnp.concatenate([pitch_bins, pad]).reshape(1, _NBINS)
    binse = jnp.concatenate([energy_bins, pad]).reshape(1, _NBINS)
    prm = _flat_params(dp) + _flat_params(pp) + _flat_params(ep)

    x3p, gidx, ld, pd, ed, tot = _run_tc(x, maskf, durf, pt, et, binsp, binse,
                                         pitch_emb, energy_emb, prm)

    gidx2 = gidx.reshape((b * _M) // _CHUNK, _CHUNK)
    out = _sc_gather(x3p.reshape(b * _LP, d), gidx2).reshape(b, _M, d)

    mel_len = jnp.minimum(tot.reshape(b), max_len)
    return (out, ld.reshape(b, l), pd.reshape(b, l), ed.reshape(b, l),
            duration_target, mel_len)


# speedup vs baseline: 1.2611x; 1.0227x over previous
import functools

import jax
import jax.numpy as jnp
from jax import lax
from jax.experimental import pallas as pl
from jax.experimental.pallas import tpu as pltpu
from jax.experimental.pallas import tpu_sc as plsc

_L = 1024
_D = 256
_M = 4096
_NBINS = 256
_LP = _L + 8

_HI = jax.lax.Precision.DEFAULT


def _dot(a, b):
    return jnp.dot(a, b, precision=_HI, preferred_element_type=jnp.float32)


def _conv3(xb, w_ref):
    y0 = _dot(xb, w_ref[0])
    y1 = _dot(xb, w_ref[1])
    y2 = _dot(xb, w_ref[2])
    z = jnp.zeros((1, xb.shape[1]), jnp.float32)
    y0s = jnp.concatenate([z, y0[:-1, :]], axis=0)
    y2s = jnp.concatenate([y2[1:, :], z], axis=0)
    return y0s + y1 + y2s


def _count_le_bf16(sel_bool):
    sel = sel_bool.astype(jnp.bfloat16)
    ones = jnp.ones((sel.shape[1], 1), jnp.bfloat16)
    return jnp.dot(sel, ones, preferred_element_type=jnp.float32)


def _ln(h, g, b):
    m = jnp.mean(h, axis=-1, keepdims=True)
    e = h - m
    v = jnp.mean(e * e, axis=-1, keepdims=True)
    return e * jax.lax.rsqrt(v + 1e-5) * g + b


def _predict(xb, mask_keep, w1, b1, g1, be1, w2, b2, g2, be2, wl, bl):
    h = jax.nn.relu(_conv3(xb, w1) + b1[...])
    h = _ln(h, g1[...], be1[...])
    h = jax.nn.relu(_conv3(h, w2) + b2[...])
    h = _ln(h, g2[...], be2[...])
    o = _dot(h, wl[...]) + bl[...]
    return o * mask_keep


def _emb_add(vcol, bins_row, emb_ref):
    cnt = _count_le_bf16(bins_row < vcol)
    lane = jax.lax.broadcasted_iota(jnp.int32, (vcol.shape[0], _NBINS), 1)
    onehot = (cnt == lane.astype(jnp.float32)).astype(jnp.float32)
    return _dot(onehot, emb_ref[...])


def _body(x_ref, maskf_ref, durf_ref, pt_ref, et_ref, binsp_ref, binse_ref,
          pemb_ref, eemb_ref, *rest):
    prm, (x3_ref, gidx_ref, ld_ref, pd_ref, ed_ref, tot_ref) = rest[:30], rest[30:]
    dp_p, pp_p, ep_p = prm[0:10], prm[10:20], prm[20:30]

    xb = x_ref[0]
    mask_keep = 1.0 - maskf_ref[0]

    ld_ref[0] = _predict(xb, mask_keep, *dp_p)
    pd_ref[0] = _predict(xb, mask_keep, *pp_p)
    x2 = xb + _emb_add(pt_ref[0], binsp_ref[...], pemb_ref)
    ed_ref[0] = _predict(x2, mask_keep, *ep_p)
    x3 = x2 + _emb_add(et_ref[0], binse_ref[...], eemb_ref)

    x3_ref[0, : _L, :] = x3
    x3_ref[0, _L:, :] = jnp.zeros((_LP - _L, _D), jnp.float32)

    d_row = durf_ref[0].astype(jnp.float32)
    ii = jax.lax.broadcasted_iota(jnp.int32, (_L, _L), 0)
    jj = jax.lax.broadcasted_iota(jnp.int32, (_L, _L), 1)
    cum = jnp.dot(d_row.astype(jnp.bfloat16),
                  (ii <= jj).astype(jnp.bfloat16),
                  preferred_element_type=jnp.float32)
    tot_ref[...] = jnp.sum(d_row).astype(jnp.int32).reshape(1, 1, 1)

    base = pl.program_id(0) * _LP
    chunk = 1024
    for c in range(_M // chunk):
        i0 = min((1024 * c // 7) // 128 * 128, _L)
        tcol = (jax.lax.broadcasted_iota(jnp.int32, (chunk, 1), 0)
                + c * chunk).astype(jnp.float32)
        cnt = _count_le_bf16(cum[:, i0:] <= tcol)
        gidx_ref[0, pl.ds(c * chunk, chunk), :] = (
            cnt.astype(jnp.int32) + (base + i0))


@jax.jit
def _run_tc(x, maskf, durf, pt, et, binsp, binse, pemb, eemb, prm):
    b = x.shape[0]
    row_spec = pl.BlockSpec((1, _L, 1), lambda i: (i, 0, 0))
    full2 = lambda a: pl.BlockSpec(a.shape, lambda i: (0,) * a.ndim)
    in_specs = [
        pl.BlockSpec((1, _L, _D), lambda i: (i, 0, 0)),
        row_spec,
        pl.BlockSpec((1, 1, _L), lambda i: (i, 0, 0)),
        row_spec,
        row_spec,
        full2(binsp), full2(binse), full2(pemb), full2(eemb),
    ] + [full2(p) for p in prm]
    out_specs = (
        pl.BlockSpec((1, _LP, _D), lambda i: (i, 0, 0)),
        pl.BlockSpec((1, _M, 1), lambda i: (i, 0, 0)),
        row_spec, row_spec, row_spec,
        pl.BlockSpec((1, 1, 1), lambda i: (i, 0, 0)),
    )
    out_shape = (
        jax.ShapeDtypeStruct((b, _LP, _D), jnp.float32),
        jax.ShapeDtypeStruct((b, _M, 1), jnp.int32),
        jax.ShapeDtypeStruct((b, _L, 1), jnp.float32),
        jax.ShapeDtypeStruct((b, _L, 1), jnp.float32),
        jax.ShapeDtypeStruct((b, _L, 1), jnp.float32),
        jax.ShapeDtypeStruct((b, 1, 1), jnp.int32),
    )
    return pl.pallas_call(
        _body,
        grid=(b,),
        in_specs=in_specs,
        out_specs=out_specs,
        out_shape=out_shape,
    )(x, maskf, durf, pt, et, binsp, binse, pemb, eemb, *prm)


_CHUNK = 64
_NBUF = 6


@jax.jit
def _sc_gather(table, gidx2):
    info = plsc.get_sparse_core_info()
    nc, ns = info.num_cores, info.num_subcores
    nw = nc * ns
    total_chunks = gidx2.shape[0]
    n_chunks = total_chunks // nw
    rows_total = total_chunks * _CHUNK
    mesh = plsc.VectorSubcoreMesh(core_axis_name="c", subcore_axis_name="s")

    @functools.partial(
        pl.kernel, mesh=mesh,
        out_type=jax.ShapeDtypeStruct((rows_total, _D), jnp.float32),
        scratch_types=[
            pltpu.VMEM((n_chunks, _CHUNK), jnp.int32),
            pltpu.VMEM((_NBUF, _CHUNK, _D), jnp.float32),
        ] + [pltpu.SemaphoreType.DMA] * (2 * _NBUF),
    )
    def k(table_hbm, gidx_hbm, out_hbm, idx_v, rows_v, *sems):
        cc = lax.axis_index("c")
        ss_i = lax.axis_index("s")
        wid = ss_i * nc + (cc + ss_i) % nc
        gs, ss = list(sems[:_NBUF]), list(sems[_NBUF:])
        gat = [None] * _NBUF
        sca = [None] * _NBUF

        base_chunk = wid * n_chunks
        pltpu.sync_copy(gidx_hbm.at[pl.ds(base_chunk, n_chunks)], idx_v)

        def fire(g):
            bb = g % _NBUF
            if sca[bb] is not None:
                sca[bb].wait()
                sca[bb] = None
            gat[bb] = pltpu.async_copy(
                table_hbm.at[idx_v.at[g]], rows_v.at[bb], gs[bb])

        depth = min(_NBUF - 1, n_chunks)
        for g in range(depth):
            fire(g)
        for g in range(n_chunks):
            bb = g % _NBUF
            gat[bb].wait()
            gat[bb] = None
            if g + depth < n_chunks:
                fire(g + depth)
            sca[bb] = pltpu.async_copy(
                rows_v.at[bb],
                out_hbm.at[pl.ds((base_chunk + g) * _CHUNK, _CHUNK)],
                ss[bb])
        for s in sca:
            if s is not None:
                s.wait()

    return k(table, gidx2)


_PKEYS = ("w1", "b1", "g1", "be1", "w2", "b2", "g2", "be2", "wl", "bl")


def _flat_params(p):
    out = []
    for k in _PKEYS:
        a = p[k]
        if k in ("b1", "g1", "be1", "b2", "g2", "be2"):
            a = a.reshape(1, _D)
        elif k == "bl":
            a = a.reshape(1, 1)
        out.append(a)
    return out


def kernel(x, src_mask, duration_target, pitch_target, energy_target, max_len,
           dp, pp, ep, pitch_bins, energy_bins, pitch_emb, energy_emb):
    b, l, d = x.shape
    maskf = src_mask.astype(jnp.float32).reshape(b, l, 1)
    durf = duration_target.reshape(b, 1, l)
    pt = pitch_target.reshape(b, l, 1)
    et = energy_target.reshape(b, l, 1)
    pad = jnp.full((1,), jnp.inf, jnp.float32)
    binsp = jnp.concatenate([pitch_bins, pad]).reshape(1, _NBINS)
    binse = jnp.concatenate([energy_bins, pad]).reshape(1, _NBINS)
    prm = _flat_params(dp) + _flat_params(pp) + _flat_params(ep)

    x3p, gidx, ld, pd, ed, tot = _run_tc(x, maskf, durf, pt, et, binsp, binse,
                                         pitch_emb, energy_emb, prm)

    gidx2 = gidx.reshape((b * _M) // _CHUNK, _CHUNK)
    out = _sc_gather(x3p.reshape(b * _LP, d), gidx2).reshape(b, _M, d)

    mel_len = jnp.minimum(tot.reshape(b), max_len)
    return (out, ld.reshape(b, l), pd.reshape(b, l), ed.reshape(b, l),
            duration_target, mel_len)
